# baseline (speedup 1.0000x reference)
import functools

import jax
import jax.numpy as jnp
from jax import lax
from jax.experimental import pallas as pl
from jax.experimental.pallas import tpu as pltpu
from jax.experimental.pallas import tpu_sc as plsc

N_IN = 10000
N_OUT = 10000
E = 320000
C = 128
DW = 144
NC = 2
NS = 16
NW = NC * NS
CHUNK = 48
RPT0 = 264
RPT1 = 156
RPTMAX = max(RPT0, RPT1)
ROWS_PAD = NS * (RPT0 + RPT1)
E_PAD = ROWS_PAD * CHUNK
N_TAB = N_IN + 16
ZSRC = N_IN
ACC_ROWS = N_OUT
ZPT = N_OUT // NS


def _prep_body(x_ref, w_ref, b_ref, attn_ref, alpha_ref, out_ref):
    xb = x_ref[...]
    wxT = lax.dot_general(xb, w_ref[...], (((0,), (1,)), ((), ())),
                          preferred_element_type=jnp.float32)
    wxT = wxT + b_ref[...]
    a = jnp.dot(wxT, attn_ref[...],
                preferred_element_type=jnp.float32)
    alpha = alpha_ref[...]
    g = jnp.exp(jnp.where(a >= 0.0, a, alpha * a))
    out_ref[0:N_IN, 0:C] = wxT * g
    lane = lax.broadcasted_iota(jnp.int32, (N_IN, DW - C), 1)
    out_ref[0:N_IN, C:DW] = jnp.where(lane == 0, g, 0.0)
    out_ref[N_IN:N_TAB, :] = jnp.zeros((N_TAB - N_IN, DW), jnp.float32)


def _prep(x, W, b, attn_w, prelu_alpha):
    return pl.pallas_call(
        _prep_body,
        out_shape=jax.ShapeDtypeStruct((N_TAB, DW), jnp.float32),
    )(x, W, b.reshape(1, C), attn_w.reshape(C, 1),
      jnp.reshape(prelu_alpha, (1, 1)))


@functools.partial(
    pl.kernel,
    mesh=plsc.VectorSubcoreMesh(core_axis_name="c", subcore_axis_name="s"),
    out_type=jax.ShapeDtypeStruct((NC, N_OUT, DW), jnp.float32),
    compiler_params=pltpu.CompilerParams(use_tc_tiling_on_sc=False),
    scratch_types=[
        pltpu.VMEM((RPTMAX, CHUNK), jnp.int32),
        pltpu.VMEM((6, CHUNK), jnp.int32),
        pltpu.VMEM((6, CHUNK), jnp.int32),
        pltpu.VMEM((3, CHUNK, DW), jnp.float32),
        pltpu.VMEM_SHARED((ACC_ROWS, DW), jnp.float32),
        [pltpu.SemaphoreType.DMA] * 3,
        [pltpu.SemaphoreType.DMA] * 3,
    ],
)
def _sc_scatter(pk_hbm, h_hbm, z_hbm, out_hbm,
                pk_v, src_v, tgt_v, rows_v, acc, sem_g, sem_s):
    c = lax.axis_index("c")
    s = lax.axis_index("s")
    rpt = jnp.where(c == 0, RPT0, RPT1)

    @pl.when(c == 0)
    def _():
        pltpu.sync_copy(pk_hbm.at[pl.ds(s * RPT0, RPT0)],
                        pk_v.at[pl.ds(0, RPT0)])

    @pl.when(c == 1)
    def _():
        pltpu.sync_copy(pk_hbm.at[pl.ds(NS * RPT0 + s * RPT1, RPT1)],
                        pk_v.at[pl.ds(0, RPT1)])

    pltpu.sync_copy(z_hbm, acc.at[pl.ds(s * ZPT, ZPT)])
    plsc.subcore_barrier()

    def unpack(r, u):
        for q in range(CHUNK // 16):
            pk = pk_v[r, pl.ds(q * 16, 16)]
            src_v[u, pl.ds(q * 16, 16)] = pk & 0xFFFF
            tgt_v[u, pl.ds(q * 16, 16)] = lax.shift_right_logical(pk, 16)

    unpack(0, 0)
    pltpu.async_copy(h_hbm.at[src_v.at[0]], rows_v.at[0], sem_g[0])
    unpack(1, 1)
    pltpu.async_copy(h_hbm.at[src_v.at[1]], rows_v.at[1], sem_g[1])
    unpack(2, 2)
    unpack(3, 3)

    def step(r, k):
        b = k % 3
        u2 = (k + 2) % 6
        u4 = (k + 4) % 6

        @pl.when(r + 4 < rpt)
        def _():
            unpack(r + 4, u4)

        pltpu.make_async_copy(h_hbm.at[src_v.at[k]], rows_v.at[b],
                              sem_g[b]).wait()
        pltpu.async_copy(rows_v.at[b], acc.at[tgt_v.at[k]], sem_s[b],
                         add=True)
        b2 = (b + 2) % 3
        u5 = (k + 5) % 6

        @pl.when(r >= 1)
        def _():
            pltpu.make_async_copy(rows_v.at[b2], acc.at[tgt_v.at[u5]],
                                  sem_s[b2]).wait()

        @pl.when(r + 2 < rpt)
        def _():
            pltpu.async_copy(h_hbm.at[src_v.at[u2]], rows_v.at[b2],
                             sem_g[b2])

    def body(i, carry):
        r0 = 6 * i
        for k in range(6):
            step(r0 + k, k)
        return carry

    lax.fori_loop(0, rpt // 6, body, 0)
    pltpu.make_async_copy(rows_v.at[2], acc.at[tgt_v.at[5]],
                          sem_s[2]).wait()
    plsc.subcore_barrier()
    pltpu.sync_copy(acc.at[pl.ds(s * ZPT, ZPT)],
                    out_hbm.at[c, pl.ds(s * ZPT, ZPT)])


def _fin_body(p0_ref, p1_ref, out_ref):
    t = p0_ref[...] + p1_ref[...]
    num = t[:, 0:C]
    den = t[:, C:C + 1]
    den = jnp.where(den == 0.0, 1.0, den)
    out_ref[...] = (num / den).T


def _fin(p0, p1):
    return pl.pallas_call(
        _fin_body,
        out_shape=jax.ShapeDtypeStruct((C, N_OUT), jnp.float32),
    )(p0, p1)


def kernel(x, W, b, attn_w, prelu_alpha, edges):
    e = edges.astype(jnp.int32)
    tgt = e[:, 0]
    src = e[:, 1]
    pad = E_PAD - E
    packed = tgt * 65536 + src
    pk = jnp.concatenate(
        [packed, jnp.full((pad,), ZSRC, jnp.int32)]).reshape(ROWS_PAD, CHUNK)
    haug = _prep(x, W, b, attn_w, prelu_alpha)
    zeros = jnp.zeros((ZPT, DW), jnp.float32)
    partials = _sc_scatter(pk, haug, zeros)
    return _fin(partials[0], partials[1])


# speedup vs baseline: 1.0246x; 1.0246x over previous
import functools

import jax
import jax.numpy as jnp
from jax import lax
from jax.experimental import pallas as pl
from jax.experimental.pallas import tpu as pltpu
from jax.experimental.pallas import tpu_sc as plsc

N_IN = 10000
N_OUT = 10000
E = 320000
C = 128
DW = 144
NC = 2
NS = 16
NW = NC * NS
CHUNK = 48
RPT0 = 264
RPT1 = 156
RPTMAX = max(RPT0, RPT1)
ROWS_PAD = NS * (RPT0 + RPT1)
E_PAD = ROWS_PAD * CHUNK
N_TAB = N_IN + 16
ZSRC = N_IN
ACC_ROWS = N_OUT
ZPT = N_OUT // NS


def _prep_body(x_ref, w_ref, b_ref, attn_ref, alpha_ref, out_ref):
    xb = x_ref[...]
    wxT = lax.dot_general(xb, w_ref[...], (((0,), (1,)), ((), ())),
                          preferred_element_type=jnp.float32)
    wxT = wxT + b_ref[...]
    a = jnp.dot(wxT, attn_ref[...],
                preferred_element_type=jnp.float32)
    alpha = alpha_ref[...]
    g = jnp.exp(jnp.where(a >= 0.0, a, alpha * a))
    out_ref[0:N_IN, 0:C] = wxT * g
    lane = lax.broadcasted_iota(jnp.int32, (N_IN, DW - C), 1)
    out_ref[0:N_IN, C:DW] = jnp.where(lane == 0, g, 0.0)
    out_ref[N_IN:N_TAB, :] = jnp.zeros((N_TAB - N_IN, DW), jnp.float32)


def _prep(x, W, b, attn_w, prelu_alpha):
    return pl.pallas_call(
        _prep_body,
        out_shape=jax.ShapeDtypeStruct((N_TAB, DW), jnp.float32),
    )(x, W, b.reshape(1, C), attn_w.reshape(C, 1),
      jnp.reshape(prelu_alpha, (1, 1)))


@functools.partial(
    pl.kernel,
    mesh=plsc.VectorSubcoreMesh(core_axis_name="c", subcore_axis_name="s"),
    out_type=jax.ShapeDtypeStruct((NC, N_OUT, DW), jnp.float32),
    compiler_params=pltpu.CompilerParams(use_tc_tiling_on_sc=False),
    scratch_types=[
        pltpu.VMEM((RPTMAX, CHUNK), jnp.int32),
        pltpu.VMEM((6, CHUNK), jnp.int32),
        pltpu.VMEM((6, CHUNK), jnp.int32),
        pltpu.VMEM((3, CHUNK, DW), jnp.float32),
        pltpu.VMEM_SHARED((ACC_ROWS, DW), jnp.float32),
        [pltpu.SemaphoreType.DMA] * 3,
        [pltpu.SemaphoreType.DMA] * 3,
    ],
)
def _sc_scatter(pk_hbm, h_hbm, z_hbm, out_hbm,
                pk_v, src_v, tgt_v, rows_v, acc, sem_g, sem_s):
    c = lax.axis_index("c")
    s = lax.axis_index("s")
    rpt = jnp.where(c == 0, RPT0, RPT1)

    @pl.when(c == 0)
    def _():
        pltpu.sync_copy(pk_hbm.at[pl.ds(s * RPT0, RPT0)],
                        pk_v.at[pl.ds(0, RPT0)])

    @pl.when(c == 1)
    def _():
        pltpu.sync_copy(pk_hbm.at[pl.ds(NS * RPT0 + s * RPT1, RPT1)],
                        pk_v.at[pl.ds(0, RPT1)])

    pltpu.sync_copy(z_hbm, acc.at[pl.ds(s * ZPT, ZPT)])
    plsc.subcore_barrier()

    def unpack(r, u):
        for q in range(CHUNK // 16):
            pk = pk_v[r, pl.ds(q * 16, 16)]
            src_v[u, pl.ds(q * 16, 16)] = pk & 0xFFFF
            tgt_v[u, pl.ds(q * 16, 16)] = lax.shift_right_logical(pk, 16)

    unpack(0, 0)
    pltpu.async_copy(h_hbm.at[src_v.at[0]], rows_v.at[0], sem_g[0])
    unpack(1, 1)
    pltpu.async_copy(h_hbm.at[src_v.at[1]], rows_v.at[1], sem_g[1])
    unpack(2, 2)
    unpack(3, 3)

    def step(r, k):
        b = k % 3
        u2 = (k + 2) % 6
        u4 = (k + 4) % 6

        @pl.when(r + 4 < rpt)
        def _():
            unpack(r + 4, u4)

        pltpu.make_async_copy(h_hbm.at[src_v.at[k]], rows_v.at[b],
                              sem_g[b]).wait()
        pltpu.async_copy(rows_v.at[b], acc.at[tgt_v.at[k]], sem_s[b],
                         add=True)
        b2 = (b + 2) % 3
        u5 = (k + 5) % 6

        @pl.when(r >= 1)
        def _():
            pltpu.make_async_copy(rows_v.at[b2], acc.at[tgt_v.at[u5]],
                                  sem_s[b2]).wait()

        @pl.when(r + 2 < rpt)
        def _():
            pltpu.async_copy(h_hbm.at[src_v.at[u2]], rows_v.at[b2],
                             sem_g[b2])

    def body(i, carry):
        r0 = 6 * i
        for k in range(6):
            step(r0 + k, k)
        return carry

    lax.fori_loop(0, rpt // 6, body, 0)
    pltpu.make_async_copy(rows_v.at[2], acc.at[tgt_v.at[5]],
                          sem_s[2]).wait()
    plsc.subcore_barrier()
    pltpu.sync_copy(acc.at[pl.ds(s * ZPT, ZPT)],
                    out_hbm.at[c, pl.ds(s * ZPT, ZPT)])


def _fin_body(p_ref, out_ref):
    t = p_ref[0, :, :] + p_ref[1, :, :]
    num = t[:, 0:C]
    den = t[:, C:C + 1]
    den = jnp.where(den == 0.0, 1.0, den)
    out_ref[...] = (num / den).T


def _fin(partials):
    return pl.pallas_call(
        _fin_body,
        out_shape=jax.ShapeDtypeStruct((C, N_OUT), jnp.float32),
    )(partials)


def kernel(x, W, b, attn_w, prelu_alpha, edges):
    e = edges.astype(jnp.int32)
    tgt = e[:, 0]
    src = e[:, 1]
    pad = E_PAD - E
    packed = tgt * 65536 + src
    pk = jnp.concatenate(
        [packed, jnp.full((pad,), ZSRC, jnp.int32)]).reshape(ROWS_PAD, CHUNK)
    haug = _prep(x, W, b, attn_w, prelu_alpha)
    zeros = jnp.zeros((ZPT, DW), jnp.float32)
    partials = _sc_scatter(pk, haug, zeros)
    return _fin(partials)


# speedup vs baseline: 1.0254x; 1.0008x over previous
import functools

import jax
import jax.numpy as jnp
from jax import lax
from jax.experimental import pallas as pl
from jax.experimental.pallas import tpu as pltpu
from jax.experimental.pallas import tpu_sc as plsc

N_IN = 10000
N_OUT = 10000
E = 320000
C = 128
DW = 144
NC = 2
NS = 16
NW = NC * NS
CHUNK = 48
RPT0 = 264
RPT1 = 156
RPTMAX = max(RPT0, RPT1)
ROWS_PAD = NS * (RPT0 + RPT1)
E_PAD = ROWS_PAD * CHUNK
N_TAB = N_IN + 16
ZSRC = N_IN
ACC_ROWS = N_OUT
ZPT = N_OUT // NS


def _prep_body(x_ref, w_ref, b_ref, attn_ref, alpha_ref, out_ref):
    xb = x_ref[...]
    wxT = lax.dot_general(xb, w_ref[...], (((0,), (1,)), ((), ())),
                          preferred_element_type=jnp.float32)
    wxT = wxT + b_ref[...]
    a = jnp.dot(wxT, attn_ref[...],
                preferred_element_type=jnp.float32)
    alpha = alpha_ref[...]
    g = jnp.exp(jnp.where(a >= 0.0, a, alpha * a))
    out_ref[0:N_IN, 0:C] = wxT * g
    lane = lax.broadcasted_iota(jnp.int32, (N_IN, DW - C), 1)
    out_ref[0:N_IN, C:DW] = jnp.where(lane == 0, g, 0.0)
    out_ref[N_IN:N_TAB, :] = jnp.zeros((N_TAB - N_IN, DW), jnp.float32)


def _prep(x, W, b, attn_w, prelu_alpha):
    return pl.pallas_call(
        _prep_body,
        out_shape=jax.ShapeDtypeStruct((N_TAB, DW), jnp.float32),
    )(x, W, b.reshape(1, C), attn_w.reshape(C, 1),
      jnp.reshape(prelu_alpha, (1, 1)))


@functools.partial(
    pl.kernel,
    mesh=plsc.VectorSubcoreMesh(core_axis_name="c", subcore_axis_name="s"),
    out_type=jax.ShapeDtypeStruct((NC, N_OUT, DW), jnp.float32),
    compiler_params=pltpu.CompilerParams(use_tc_tiling_on_sc=False),
    scratch_types=[
        pltpu.VMEM((RPTMAX * CHUNK,), jnp.int32),
        pltpu.VMEM((6, CHUNK), jnp.int32),
        pltpu.VMEM((6, CHUNK), jnp.int32),
        pltpu.VMEM((3, CHUNK, DW), jnp.float32),
        pltpu.VMEM_SHARED((ACC_ROWS, DW), jnp.float32),
        [pltpu.SemaphoreType.DMA] * 3,
        [pltpu.SemaphoreType.DMA] * 3,
    ],
)
def _sc_scatter(pk_hbm, h_hbm, z_hbm, out_hbm,
                pk_v, src_v, tgt_v, rows_v, acc, sem_g, sem_s):
    c = lax.axis_index("c")
    s = lax.axis_index("s")
    rpt = jnp.where(c == 0, RPT0, RPT1)

    @pl.when(c == 0)
    def _():
        pltpu.sync_copy(pk_hbm.at[pl.ds(s * (RPT0 * CHUNK), RPT0 * CHUNK)],
                        pk_v.at[pl.ds(0, RPT0 * CHUNK)])

    @pl.when(c == 1)
    def _():
        pltpu.sync_copy(
            pk_hbm.at[pl.ds((NS * RPT0 + s * RPT1) * CHUNK, RPT1 * CHUNK)],
            pk_v.at[pl.ds(0, RPT1 * CHUNK)])

    pltpu.sync_copy(z_hbm, acc.at[pl.ds(s * ZPT, ZPT)])
    plsc.subcore_barrier()

    def unpack(r, u):
        for q in range(CHUNK // 16):
            pk = pk_v[pl.ds(r * CHUNK + q * 16, 16)]
            src_v[u, pl.ds(q * 16, 16)] = pk & 0xFFFF
            tgt_v[u, pl.ds(q * 16, 16)] = lax.shift_right_logical(pk, 16)

    unpack(0, 0)
    pltpu.async_copy(h_hbm.at[src_v.at[0]], rows_v.at[0], sem_g[0])
    unpack(1, 1)
    pltpu.async_copy(h_hbm.at[src_v.at[1]], rows_v.at[1], sem_g[1])
    unpack(2, 2)
    unpack(3, 3)

    def step(r, k):
        b = k % 3
        u2 = (k + 2) % 6
        u4 = (k + 4) % 6

        @pl.when(r + 4 < rpt)
        def _():
            unpack(r + 4, u4)

        pltpu.make_async_copy(h_hbm.at[src_v.at[k]], rows_v.at[b],
                              sem_g[b]).wait()
        pltpu.async_copy(rows_v.at[b], acc.at[tgt_v.at[k]], sem_s[b],
                         add=True)
        b2 = (b + 2) % 3
        u5 = (k + 5) % 6

        @pl.when(r >= 1)
        def _():
            pltpu.make_async_copy(rows_v.at[b2], acc.at[tgt_v.at[u5]],
                                  sem_s[b2]).wait()

        @pl.when(r + 2 < rpt)
        def _():
            pltpu.async_copy(h_hbm.at[src_v.at[u2]], rows_v.at[b2],
                             sem_g[b2])

    def body(i, carry):
        r0 = 6 * i
        for k in range(6):
            step(r0 + k, k)
        return carry

    lax.fori_loop(0, rpt // 6, body, 0)
    pltpu.make_async_copy(rows_v.at[2], acc.at[tgt_v.at[5]],
                          sem_s[2]).wait()
    plsc.subcore_barrier()
    pltpu.sync_copy(acc.at[pl.ds(s * ZPT, ZPT)],
                    out_hbm.at[c, pl.ds(s * ZPT, ZPT)])


def _fin_body(p_ref, out_ref):
    t = p_ref[0, :, :] + p_ref[1, :, :]
    num = t[:, 0:C]
    den = t[:, C:C + 1]
    den = jnp.where(den == 0.0, 1.0, den)
    out_ref[...] = (num / den).T


def _fin(partials):
    return pl.pallas_call(
        _fin_body,
        out_shape=jax.ShapeDtypeStruct((C, N_OUT), jnp.float32),
    )(partials)


def kernel(x, W, b, attn_w, prelu_alpha, edges):
    e = edges.astype(jnp.int32)
    tgt = e[:, 0]
    src = e[:, 1]
    pad = E_PAD - E
    packed = tgt * 65536 + src
    pk = jnp.concatenate([packed, jnp.full((pad,), ZSRC, jnp.int32)])
    haug = _prep(x, W, b, attn_w, prelu_alpha)
    zeros = jnp.zeros((ZPT, DW), jnp.float32)
    partials = _sc_scatter(pk, haug, zeros)
    return _fin(partials)
